# baseline (device time: 78264 ns/iter reference)
import jax
import jax.numpy as jnp
from jax import lax
from jax.experimental import pallas as pl
from jax.experimental.pallas import tpu as pltpu

T = 1024
D = 2048
V_SHARD = 16384
BV = 1024
NBLK = V_SHARD // BV


def kernel(x, W, labels):
    labels2d = labels.reshape(T, 1)
    BH = BV // 2

    def body(x_ref, w_ref, lab_ref, out_ref,
             x8_ref, sw_ref, llw_ref, stats_ref, recv_ref,
             send_sem, recv_sem):
        j = pl.program_id(0)
        my_x = lax.axis_index("x")
        my_y = lax.axis_index("y")
        my_z = lax.axis_index("z")

        @pl.when(j == 0)
        def _init():
            x8_ref[...] = x_ref[...].astype(jnp.float8_e4m3fn)
            sw_ref[...] = jnp.zeros_like(sw_ref)
            llw_ref[...] = jnp.zeros_like(llw_ref)

        base = my_x * V_SHARD + j * BV
        cols = lax.broadcasted_iota(jnp.int32, (T, BH), 1)
        for h in (0, 1):
            wq = w_ref[:, h * BH : (h + 1) * BH].astype(jnp.float8_e4m3fn)
            logits = jnp.dot(
                x8_ref[...], wq, preferred_element_type=jnp.float32
            )
            sw_ref[:, h * BH : (h + 1) * BH] += jnp.exp(logits)
            col = lab_ref[...] - (base + h * BH)
            llw_ref[:, h * BH : (h + 1) * BH] += jnp.where(
                cols == col, logits, 0.0
            )

        @pl.when(j == NBLK - 1)
        def _exchange():
            stats_ref[0] = jnp.sum(sw_ref[...], axis=1, keepdims=True)
            stats_ref[1] = jnp.sum(llw_ref[...], axis=1, keepdims=True)

            partner = (1 - my_x, my_y, my_z)
            bsem = pltpu.get_barrier_semaphore()
            pl.semaphore_signal(
                bsem, inc=1, device_id=partner,
                device_id_type=pl.DeviceIdType.MESH,
            )
            pl.semaphore_wait(bsem, 1)

            rdma = pltpu.make_async_remote_copy(
                src_ref=stats_ref,
                dst_ref=recv_ref,
                send_sem=send_sem,
                recv_sem=recv_sem,
                device_id=partner,
                device_id_type=pl.DeviceIdType.MESH,
            )
            rdma.start()
            rdma.wait()

            s_tot = stats_ref[0] + recv_ref[0]
            ll_tot = stats_ref[1] + recv_ref[1]
            out_ref[...] = jnp.log(s_tot) - ll_tot

    out = pl.pallas_call(
        body,
        grid=(NBLK,),
        out_shape=jax.ShapeDtypeStruct((T, 1), jnp.float32),
        in_specs=[
            pl.BlockSpec((T, D), lambda j: (0, 0)),
            pl.BlockSpec((D, BV), lambda j: (0, j)),
            pl.BlockSpec((T, 1), lambda j: (0, 0)),
        ],
        out_specs=pl.BlockSpec((T, 1), lambda j: (0, 0)),
        scratch_shapes=[
            pltpu.VMEM((T, D), jnp.float8_e4m3fn),
            pltpu.VMEM((T, BV), jnp.float32),
            pltpu.VMEM((T, BV), jnp.float32),
            pltpu.VMEM((2, T, 1), jnp.float32),
            pltpu.VMEM((2, T, 1), jnp.float32),
            pltpu.SemaphoreType.DMA,
            pltpu.SemaphoreType.DMA,
        ],
        compiler_params=pltpu.CompilerParams(
            collective_id=0,
            dimension_semantics=("arbitrary",),
        ),
    )(x, W, labels2d)
    return out.reshape(T)


# device time: 66065 ns/iter; 1.1847x vs baseline; 1.1847x over previous
import jax
import jax.numpy as jnp
from jax import lax
from jax.experimental import pallas as pl
from jax.experimental.pallas import tpu as pltpu

T = 1024
D = 2048
V_SHARD = 16384
VQ = 4096
BV = 1024
NBLK = VQ // BV


def kernel(x, W, labels):
    labels2d = labels.reshape(T, 1)
    quarter = (lax.axis_index("y") * 2 + lax.axis_index("z")).astype(jnp.int32)

    def body(q_ref, x_ref, w_ref, lab_ref, out_ref,
             x8_ref, sw_ref, llw_ref, stats_ref, recv_ref,
             send_sems, recv_sems):
        j = pl.program_id(0)
        my_x = lax.axis_index("x")
        my_y = lax.axis_index("y")
        my_z = lax.axis_index("z")

        @pl.when(j == 0)
        def _init():
            x8_ref[...] = x_ref[...].astype(jnp.float8_e4m3fn)
            sw_ref[...] = jnp.zeros_like(sw_ref)
            llw_ref[...] = jnp.zeros_like(llw_ref)

        logits = jnp.dot(
            x8_ref[...],
            w_ref[...].astype(jnp.float8_e4m3fn),
            preferred_element_type=jnp.float32,
        )

        sw_ref[...] += jnp.exp(logits)

        base = my_x * V_SHARD + q_ref[0] * VQ + j * BV
        col = lab_ref[...] - base
        cols = lax.broadcasted_iota(jnp.int32, (T, BV), 1)
        llw_ref[...] += jnp.where(cols == col, logits, 0.0)

        @pl.when(j == NBLK - 1)
        def _allreduce():
            stats_ref[0] = jnp.sum(sw_ref[...], axis=1, keepdims=True)
            stats_ref[1] = jnp.sum(llw_ref[...], axis=1, keepdims=True)

            z_nbr = (my_x, my_y, 1 - my_z)
            y_nbr = (my_x, 1 - my_y, my_z)
            x_nbr = (1 - my_x, my_y, my_z)

            bsem = pltpu.get_barrier_semaphore()
            for nbr in (z_nbr, y_nbr, x_nbr):
                pl.semaphore_signal(
                    bsem, inc=1, device_id=nbr,
                    device_id_type=pl.DeviceIdType.MESH,
                )
            pl.semaphore_wait(bsem, 3)

            for ph, nbr in enumerate((z_nbr, y_nbr, x_nbr)):
                rdma = pltpu.make_async_remote_copy(
                    src_ref=stats_ref,
                    dst_ref=recv_ref.at[ph],
                    send_sem=send_sems.at[ph],
                    recv_sem=recv_sems.at[ph],
                    device_id=nbr,
                    device_id_type=pl.DeviceIdType.MESH,
                )
                rdma.start()
                rdma.wait()
                stats_ref[0] = stats_ref[0] + recv_ref[ph, 0]
                stats_ref[1] = stats_ref[1] + recv_ref[ph, 1]

            out_ref[...] = jnp.log(stats_ref[0]) - stats_ref[1]

    grid_spec = pltpu.PrefetchScalarGridSpec(
        num_scalar_prefetch=1,
        grid=(NBLK,),
        in_specs=[
            pl.BlockSpec((T, D), lambda j, q: (0, 0)),
            pl.BlockSpec((D, BV), lambda j, q: (0, q[0] * NBLK + j)),
            pl.BlockSpec((T, 1), lambda j, q: (0, 0)),
        ],
        out_specs=pl.BlockSpec((T, 1), lambda j, q: (0, 0)),
        scratch_shapes=[
            pltpu.VMEM((T, D), jnp.float8_e4m3fn),
            pltpu.VMEM((T, BV), jnp.float32),
            pltpu.VMEM((T, BV), jnp.float32),
            pltpu.VMEM((2, T, 1), jnp.float32),
            pltpu.VMEM((3, 2, T, 1), jnp.float32),
            pltpu.SemaphoreType.DMA((3,)),
            pltpu.SemaphoreType.DMA((3,)),
        ],
    )

    out = pl.pallas_call(
        body,
        grid_spec=grid_spec,
        out_shape=jax.ShapeDtypeStruct((T, 1), jnp.float32),
        compiler_params=pltpu.CompilerParams(
            collective_id=0,
            dimension_semantics=("arbitrary",),
        ),
    )(quarter.reshape(1), x, W, labels2d)
    return out.reshape(T)


# device time: 29504 ns/iter; 2.6527x vs baseline; 2.2392x over previous
import jax
import jax.numpy as jnp
from jax import lax
from jax.experimental import pallas as pl
from jax.experimental.pallas import tpu as pltpu

T = 1024
D = 2048
V_SHARD = 16384
VQ = 4096
BV = 1024
NBLK = VQ // BV


def kernel(x, W, labels):
    labels2d = labels.reshape(T, 1)
    quarter = (lax.axis_index("y") * 2 + lax.axis_index("z")).astype(jnp.int32)

    def body(q_ref, x_ref, w_ref, lab_ref, out_ref,
             x8_ref, sw_ref, llw_ref, stats_ref, recv_ref,
             send_sems, recv_sems):
        j = pl.program_id(0)
        my_x = lax.axis_index("x")
        my_y = lax.axis_index("y")
        my_z = lax.axis_index("z")
        peers = [
            (my_x ^ a, my_y ^ b, my_z ^ c)
            for a in (0, 1) for b in (0, 1) for c in (0, 1)
            if (a, b, c) != (0, 0, 0)
        ]
        bsem = pltpu.get_barrier_semaphore()

        @pl.when(j == 0)
        def _init():
            x8_ref[...] = x_ref[...].astype(jnp.float8_e4m3fn)
            sw_ref[...] = jnp.zeros_like(sw_ref)
            llw_ref[...] = jnp.zeros_like(llw_ref)
            for nbr in peers:
                pl.semaphore_signal(
                    bsem, inc=1, device_id=nbr,
                    device_id_type=pl.DeviceIdType.MESH,
                )

        logits = jnp.dot(
            x8_ref[...],
            w_ref[...].astype(jnp.float8_e4m3fn),
            preferred_element_type=jnp.float32,
        )

        sw_ref[...] += jnp.exp(logits)

        base = my_x * V_SHARD + q_ref[0] * VQ + j * BV
        col = lab_ref[...] - base
        cols = lax.broadcasted_iota(jnp.int32, (T, BV), 1)
        llw_ref[...] += jnp.where(cols == col, logits, 0.0)

        @pl.when(j == NBLK - 1)
        def _allreduce():
            stats_ref[0:8, :] = jnp.sum(sw_ref[...], axis=1).reshape(8, 128)
            stats_ref[8:16, :] = jnp.sum(llw_ref[...], axis=1).reshape(8, 128)

            pl.semaphore_wait(bsem, 7)

            rdmas = []
            for k, nbr in enumerate(peers):
                rdma = pltpu.make_async_remote_copy(
                    src_ref=stats_ref,
                    dst_ref=recv_ref.at[k],
                    send_sem=send_sems.at[k],
                    recv_sem=recv_sems.at[k],
                    device_id=nbr,
                    device_id_type=pl.DeviceIdType.MESH,
                )
                rdma.start()
                rdmas.append(rdma)
            for rdma in rdmas:
                rdma.wait()

            tot = stats_ref[...]
            for k in range(7):
                tot = tot + recv_ref[k]
            out_ref[...] = jnp.log(tot[0:8, :]) - tot[8:16, :]

    grid_spec = pltpu.PrefetchScalarGridSpec(
        num_scalar_prefetch=1,
        grid=(NBLK,),
        in_specs=[
            pl.BlockSpec((T, D), lambda j, q: (0, 0)),
            pl.BlockSpec((D, BV), lambda j, q: (0, q[0] * NBLK + j)),
            pl.BlockSpec((T, 1), lambda j, q: (0, 0)),
        ],
        out_specs=pl.BlockSpec((8, 128), lambda j, q: (0, 0)),
        scratch_shapes=[
            pltpu.VMEM((T, D), jnp.float8_e4m3fn),
            pltpu.VMEM((T, BV), jnp.float32),
            pltpu.VMEM((T, BV), jnp.float32),
            pltpu.VMEM((16, 128), jnp.float32),
            pltpu.VMEM((7, 16, 128), jnp.float32),
            pltpu.SemaphoreType.DMA((7,)),
            pltpu.SemaphoreType.DMA((7,)),
        ],
    )

    out = pl.pallas_call(
        body,
        grid_spec=grid_spec,
        out_shape=jax.ShapeDtypeStruct((8, 128), jnp.float32),
        compiler_params=pltpu.CompilerParams(
            collective_id=0,
            dimension_semantics=("arbitrary",),
        ),
    )(quarter.reshape(1), x, W, labels2d)
    return out.reshape(T)
